# baseline (speedup 1.0000x reference)
import functools
import jax
import jax.numpy as jnp
from jax import lax
from jax.experimental import pallas as pl
from jax.experimental.pallas import tpu as pltpu
from jax.experimental.pallas import tpu_sc as plsc

_EPS = 1e-8
_ROWS = 100000
_D = 768
_NW = 32
_CHUNK = 64
_NG = _CHUNK // 16
_NCH = 49
_STRIDE = _NCH * _CHUNK
_LAST = _ROWS - _CHUNK


def _newton_rsqrt(x):
    i = lax.bitcast_convert_type(x, jnp.int32)
    i = jnp.int32(0x5F3759DF) - lax.shift_right_logical(i, 1)
    y = lax.bitcast_convert_type(i, jnp.float32)
    for _ in range(3):
        y = y * (jnp.float32(1.5) - jnp.float32(0.5) * x * y * y)
    return y


def _sc_call(x, user_embed):
    mesh = plsc.VectorSubcoreMesh(core_axis_name="c", subcore_axis_name="s")

    @functools.partial(
        pl.kernel,
        mesh=mesh,
        out_type=jax.ShapeDtypeStruct((_NW, 1, _STRIDE), jnp.float32),
        compiler_params=pltpu.CompilerParams(needs_layout_passes=False),
        scratch_types=[
            pltpu.VMEM((_D,), jnp.float32),
            pltpu.VMEM((_CHUNK, _D), jnp.float32),
            pltpu.VMEM((_CHUNK, _D), jnp.float32),
            pltpu.VMEM((1, _STRIDE), jnp.float32),
            pltpu.SemaphoreType.DMA,
            pltpu.SemaphoreType.DMA,
        ],
    )
    def k(x_hbm, u_hbm, out_hbm, u_v, buf0, buf1, sim_v, sem0, sem1):
        wid = lax.axis_index("s") * 2 + lax.axis_index("c")
        base = wid * _STRIDE

        pltpu.sync_copy(u_hbm.at[0], u_v)

        lane = lax.iota(jnp.int32, 16)

        nu_acc = jnp.zeros((16,), jnp.float32)
        for j in range(_D // 16):
            uv = u_v[pl.ds(j * 16, 16)]
            nu_acc = nu_acc + uv * uv
        dnums = lax.GatherDimensionNumbers(
            offset_dims=(), collapsed_slice_dims=(0,), start_index_map=(0,)
        )
        for sh in (1, 2, 4, 8):
            perm = jnp.bitwise_xor(lane, sh).reshape(16, 1)
            nu_acc = nu_acc + lax.gather(
                nu_acc, perm, dnums, (1,),
                mode=lax.GatherScatterMode.PROMISE_IN_BOUNDS,
            )
        nu2 = nu_acc
        nu = nu2 * _newton_rsqrt(jnp.maximum(nu2, jnp.float32(1e-30)))
        inv_nu = jnp.float32(1.0) / jnp.maximum(nu, jnp.float32(_EPS))

        bufs = (buf0, buf1)
        sems = (sem0, sem1)

        def dma(ci):
            start = pl.multiple_of(
                jnp.minimum(base + ci * _CHUNK, _LAST), _CHUNK
            )
            return pltpu.make_async_copy(
                x_hbm.at[pl.ds(start, _CHUNK)],
                bufs[ci % 2],
                sems[ci % 2],
            )

        dma(0).start()

        zero16 = jnp.zeros((16,), jnp.float32)
        idx_r = [lane + jnp.int32(g * 16) for g in range(_NG)]

        for ci in range(_NCH):
            if ci + 1 < _NCH:
                dma(ci + 1).start()
            dma(ci).wait()
            buf = bufs[ci % 2]
            off = base + ci * _CHUNK - jnp.minimum(base + ci * _CHUNK, _LAST)
            idx_c = [
                jnp.minimum(idx_r[g] + off, jnp.int32(_CHUNK - 1))
                for g in range(_NG)
            ]

            def feat_body(c, carry, buf=buf, idx_c=idx_c):
                accs = list(carry)
                cvec = jnp.broadcast_to(c, (16,))
                us = plsc.load_gather(u_v, [cvec])
                for g in range(_NG):
                    xv = plsc.load_gather(buf, [idx_c[g], cvec])
                    accs[2 * g] = accs[2 * g] + xv * us
                    accs[2 * g + 1] = accs[2 * g + 1] + xv * xv
                return tuple(accs)

            accs = lax.fori_loop(
                0, _D, feat_body, (zero16,) * (2 * _NG), unroll=2
            )
            for g in range(_NG):
                dacc = accs[2 * g]
                nacc = accs[2 * g + 1]
                rs = _newton_rsqrt(jnp.maximum(nacc, jnp.float32(1e-30)))
                nx = nacc * rs
                sim = dacc * inv_nu / jnp.maximum(nx, jnp.float32(_EPS))
                sim_v[0, pl.ds(ci * _CHUNK + g * 16, 16)] = sim

        pltpu.sync_copy(sim_v, out_hbm.at[wid])

    return k(x, user_embed)


def kernel(x, user_embed):
    out = _sc_call(x, user_embed)
    return out.reshape(-1)[:_ROWS]


# speedup vs baseline: 1.1175x; 1.1175x over previous
import functools
import jax
import jax.numpy as jnp
from jax import lax
from jax.experimental import pallas as pl
from jax.experimental.pallas import tpu as pltpu
from jax.experimental.pallas import tpu_sc as plsc

_EPS = 1e-8
_ROWS = 100000
_D = 768
_NW = 32
_CHUNK = 64
_NG = _CHUNK // 16
_NCH = 49
_STRIDE = _NCH * _CHUNK
_LAST = _ROWS - _CHUNK


def _newton_rsqrt(x):
    i = lax.bitcast_convert_type(x, jnp.int32)
    i = jnp.int32(0x5F3759DF) - lax.shift_right_logical(i, 1)
    y = lax.bitcast_convert_type(i, jnp.float32)
    for _ in range(3):
        y = y * (jnp.float32(1.5) - jnp.float32(0.5) * x * y * y)
    return y


def _sc_call(x, user_embed):
    mesh = plsc.VectorSubcoreMesh(core_axis_name="c", subcore_axis_name="s")

    @functools.partial(
        pl.kernel,
        mesh=mesh,
        out_type=jax.ShapeDtypeStruct((_NW, 1, _STRIDE), jnp.float32),
        compiler_params=pltpu.CompilerParams(needs_layout_passes=False),
        scratch_types=[
            pltpu.VMEM((_D,), jnp.float32),
            pltpu.VMEM((_CHUNK, _D), jnp.float32),
            pltpu.VMEM((_CHUNK, _D), jnp.float32),
            pltpu.VMEM((1, _STRIDE), jnp.float32),
            pltpu.SemaphoreType.DMA,
            pltpu.SemaphoreType.DMA,
        ],
    )
    def k(x_hbm, u_hbm, out_hbm, u_v, buf0, buf1, sim_v, sem0, sem1):
        wid = lax.axis_index("s") * 2 + lax.axis_index("c")
        base = wid * _STRIDE

        pltpu.sync_copy(u_hbm.at[0], u_v)

        lane = lax.iota(jnp.int32, 16)

        nu_acc = jnp.zeros((16,), jnp.float32)
        for j in range(_D // 16):
            uv = u_v[pl.ds(j * 16, 16)]
            nu_acc = nu_acc + uv * uv
        dnums = lax.GatherDimensionNumbers(
            offset_dims=(), collapsed_slice_dims=(0,), start_index_map=(0,)
        )
        for sh in (1, 2, 4, 8):
            perm = jnp.bitwise_xor(lane, sh).reshape(16, 1)
            nu_acc = nu_acc + lax.gather(
                nu_acc, perm, dnums, (1,),
                mode=lax.GatherScatterMode.PROMISE_IN_BOUNDS,
            )
        nu2 = nu_acc
        nu = nu2 * _newton_rsqrt(jnp.maximum(nu2, jnp.float32(1e-30)))
        inv_nu = jnp.float32(1.0) / jnp.maximum(nu, jnp.float32(_EPS))

        bufs = (buf0, buf1)
        sems = (sem0, sem1)

        def dma(ci):
            start = pl.multiple_of(
                jnp.minimum(base + ci * _CHUNK, _LAST), _CHUNK
            )
            return pltpu.make_async_copy(
                x_hbm.at[pl.ds(start, _CHUNK)],
                bufs[ci % 2],
                sems[ci % 2],
            )

        dma(0).start()

        zero16 = jnp.zeros((16,), jnp.float32)
        idx_r = [lane + jnp.int32(g * 16) for g in range(_NG)]

        for ci in range(_NCH):
            if ci + 1 < _NCH:
                dma(ci + 1).start()
            dma(ci).wait()
            buf = bufs[ci % 2]
            off = base + ci * _CHUNK - jnp.minimum(base + ci * _CHUNK, _LAST)
            idx_c = [
                jnp.minimum(idx_r[g] + off, jnp.int32(_CHUNK - 1))
                for g in range(_NG)
            ]

            def feat_body(c, carry, buf=buf, idx_c=idx_c):
                accs = list(carry[:-1])
                cvec = carry[-1]
                us = plsc.load_gather(u_v, [cvec])
                for g in range(_NG):
                    xv = plsc.load_gather(buf, [idx_c[g], cvec])
                    accs[2 * g] = accs[2 * g] + xv * us
                    accs[2 * g + 1] = accs[2 * g + 1] + xv * xv
                return tuple(accs) + (cvec + 1,)

            res = lax.fori_loop(
                0, _D, feat_body,
                (zero16,) * (2 * _NG) + (jnp.zeros((16,), jnp.int32),),
                unroll=4,
            )
            accs = res[:-1]
            for g in range(_NG):
                dacc = accs[2 * g]
                nacc = accs[2 * g + 1]
                rs = _newton_rsqrt(jnp.maximum(nacc, jnp.float32(1e-30)))
                nx = nacc * rs
                sim = dacc * inv_nu / jnp.maximum(nx, jnp.float32(_EPS))
                sim_v[0, pl.ds(ci * _CHUNK + g * 16, 16)] = sim

        pltpu.sync_copy(sim_v, out_hbm.at[wid])

    return k(x, user_embed)


def kernel(x, user_embed):
    out = _sc_call(x, user_embed)
    return out.reshape(-1)[:_ROWS]


# speedup vs baseline: 11.8743x; 10.6262x over previous
import functools
import jax
import jax.numpy as jnp
from jax import lax
from jax.experimental import pallas as pl
from jax.experimental.pallas import tpu as pltpu
from jax.experimental.pallas import tpu_sc as plsc

_EPS = 1e-8
_ROWS = 100000
_D = 768

_BLK = 4096
_TC_NBLK = 23
_TC_ROWS = _TC_NBLK * _BLK
_TC_GRID = 12

_NW = 32
_CHUNK = 64
_NG = _CHUNK // 16
_NCH = 3
_STRIDE = _NCH * _CHUNK
_LAST = _ROWS - _CHUNK
_SC_ROWS = _ROWS - _TC_ROWS


def _tc_body(u_ref, xa_ref, xb_ref, o_ref):
    i = pl.program_id(0)
    u = u_ref[0, :]
    nu = jnp.sqrt(jnp.sum(u * u))
    inv_nu = 1.0 / jnp.maximum(nu, _EPS)
    for xref, row in ((xa_ref, 2 * i), (xb_ref, 2 * i + 1)):
        x = xref[...]
        xg = x[:, 0:128]
        pd = xg * u[0:128][None, :]
        pn = xg * xg
        for g in range(1, _D // 128):
            xg = x[:, g * 128:(g + 1) * 128]
            pd = pd + xg * u[g * 128:(g + 1) * 128][None, :]
            pn = pn + xg * xg
        dot = jnp.sum(pd.T, axis=0)
        nrm = jnp.sum(pn.T, axis=0)
        sim = dot * inv_nu / jnp.maximum(jnp.sqrt(nrm), _EPS)
        o_ref[pl.ds(row, 1), :] = sim.reshape(1, _BLK)


def _tc_call(x, user_embed):
    out = pl.pallas_call(
        _tc_body,
        grid=(_TC_GRID,),
        in_specs=[
            pl.BlockSpec((1, _D), lambda i: (0, 0)),
            pl.BlockSpec((_BLK, _D), lambda i: (2 * i, 0)),
            pl.BlockSpec((_BLK, _D),
                         lambda i: (jnp.minimum(2 * i + 1, _TC_NBLK - 1), 0)),
        ],
        out_specs=pl.BlockSpec((2 * _TC_GRID, _BLK), lambda i: (0, 0)),
        out_shape=jax.ShapeDtypeStruct((2 * _TC_GRID, _BLK), jnp.float32),
    )(user_embed, x, x)
    return out.reshape(-1)[:_TC_ROWS]


def _newton_rsqrt(x):
    i = lax.bitcast_convert_type(x, jnp.int32)
    i = jnp.int32(0x5F3759DF) - lax.shift_right_logical(i, 1)
    y = lax.bitcast_convert_type(i, jnp.float32)
    for _ in range(3):
        y = y * (jnp.float32(1.5) - jnp.float32(0.5) * x * y * y)
    return y


def _sc_call(x, user_embed):
    mesh = plsc.VectorSubcoreMesh(core_axis_name="c", subcore_axis_name="s")

    @functools.partial(
        pl.kernel,
        mesh=mesh,
        out_type=jax.ShapeDtypeStruct((_NW, 1, _STRIDE), jnp.float32),
        compiler_params=pltpu.CompilerParams(needs_layout_passes=False),
        scratch_types=[
            pltpu.VMEM((_D,), jnp.float32),
            pltpu.VMEM((_CHUNK, _D), jnp.float32),
            pltpu.VMEM((_CHUNK, _D), jnp.float32),
            pltpu.VMEM((1, _STRIDE), jnp.float32),
            pltpu.SemaphoreType.DMA,
            pltpu.SemaphoreType.DMA,
        ],
    )
    def k(x_hbm, u_hbm, out_hbm, u_v, buf0, buf1, sim_v, sem0, sem1):
        wid = lax.axis_index("s") * 2 + lax.axis_index("c")
        base = _TC_ROWS + wid * _STRIDE

        pltpu.sync_copy(u_hbm.at[0], u_v)

        lane = lax.iota(jnp.int32, 16)

        nu_acc = jnp.zeros((16,), jnp.float32)
        for j in range(_D // 16):
            uv = u_v[pl.ds(j * 16, 16)]
            nu_acc = nu_acc + uv * uv
        dnums = lax.GatherDimensionNumbers(
            offset_dims=(), collapsed_slice_dims=(0,), start_index_map=(0,)
        )
        for sh in (1, 2, 4, 8):
            perm = jnp.bitwise_xor(lane, sh).reshape(16, 1)
            nu_acc = nu_acc + lax.gather(
                nu_acc, perm, dnums, (1,),
                mode=lax.GatherScatterMode.PROMISE_IN_BOUNDS,
            )
        nu2 = nu_acc
        nu = nu2 * _newton_rsqrt(jnp.maximum(nu2, jnp.float32(1e-30)))
        inv_nu = jnp.float32(1.0) / jnp.maximum(nu, jnp.float32(_EPS))

        bufs = (buf0, buf1)
        sems = (sem0, sem1)

        def dma(ci):
            start = pl.multiple_of(
                jnp.minimum(base + ci * _CHUNK, _LAST), 8
            )
            return pltpu.make_async_copy(
                x_hbm.at[pl.ds(start, _CHUNK)],
                bufs[ci % 2],
                sems[ci % 2],
            )

        dma(0).start()

        zero16 = jnp.zeros((16,), jnp.float32)
        idx_r = [lane + jnp.int32(g * 16) for g in range(_NG)]

        for ci in range(_NCH):
            if ci + 1 < _NCH:
                dma(ci + 1).start()
            dma(ci).wait()
            buf = bufs[ci % 2]
            off = base + ci * _CHUNK - jnp.minimum(base + ci * _CHUNK, _LAST)
            idx_c = [
                jnp.minimum(idx_r[g] + off, jnp.int32(_CHUNK - 1))
                for g in range(_NG)
            ]

            def feat_body(c, carry, buf=buf, idx_c=idx_c):
                accs = list(carry[:-1])
                cvec = carry[-1]
                us = plsc.load_gather(u_v, [cvec])
                for g in range(_NG):
                    xv = plsc.load_gather(buf, [idx_c[g], cvec])
                    accs[2 * g] = accs[2 * g] + xv * us
                    accs[2 * g + 1] = accs[2 * g + 1] + xv * xv
                return tuple(accs) + (cvec + 1,)

            res = lax.fori_loop(
                0, _D, feat_body,
                (zero16,) * (2 * _NG) + (jnp.zeros((16,), jnp.int32),),
                unroll=4,
            )
            accs = res[:-1]
            for g in range(_NG):
                dacc = accs[2 * g]
                nacc = accs[2 * g + 1]
                rs = _newton_rsqrt(jnp.maximum(nacc, jnp.float32(1e-30)))
                nx = nacc * rs
                sim = dacc * inv_nu / jnp.maximum(nx, jnp.float32(_EPS))
                sim_v[0, pl.ds(ci * _CHUNK + g * 16, 16)] = sim

        pltpu.sync_copy(sim_v, out_hbm.at[wid])

    return k(x, user_embed).reshape(-1)[:_SC_ROWS]


def kernel(x, user_embed):
    tc = _tc_call(x, user_embed)
    sc = _sc_call(x, user_embed)
    return jnp.concatenate([tc, sc])


# speedup vs baseline: 14.2307x; 1.1984x over previous
import jax
import jax.numpy as jnp
from jax.experimental import pallas as pl

_EPS = 1e-8
_ROWS = 100000
_D = 768
_BLK = 4096
_NBLK = 25
_GRID = 13


def _body(u_ref, xa_ref, xb_ref, o_ref):
    i = pl.program_id(0)
    u = u_ref[0, :]
    nu = jnp.sqrt(jnp.sum(u * u))
    inv_nu = 1.0 / jnp.maximum(nu, _EPS)
    for xref, row in ((xa_ref, 2 * i), (xb_ref, 2 * i + 1)):
        x = xref[...]
        xg = x[:, 0:128]
        pd = xg * u[0:128][None, :]
        pn = xg * xg
        for g in range(1, _D // 128):
            xg = x[:, g * 128:(g + 1) * 128]
            pd = pd + xg * u[g * 128:(g + 1) * 128][None, :]
            pn = pn + xg * xg
        dot = jnp.sum(pd.T, axis=0)
        nrm = jnp.sum(pn.T, axis=0)
        sim = dot * inv_nu / jnp.maximum(jnp.sqrt(nrm), _EPS)
        o_ref[pl.ds(row, 1), :] = sim.reshape(1, _BLK)


def kernel(x, user_embed):
    out = pl.pallas_call(
        _body,
        grid=(_GRID,),
        in_specs=[
            pl.BlockSpec((1, _D), lambda i: (0, 0)),
            pl.BlockSpec((_BLK, _D), lambda i: (2 * i, 0)),
            pl.BlockSpec((_BLK, _D), lambda i: (jnp.minimum(2 * i + 1, _NBLK - 1), 0)),
        ],
        out_specs=pl.BlockSpec((2 * _GRID, _BLK), lambda i: (0, 0)),
        out_shape=jax.ShapeDtypeStruct((2 * _GRID, _BLK), jnp.float32),
    )(user_embed, x, x)
    return out.reshape(-1)[:_ROWS]
